# baseline (speedup 1.0000x reference)
import math

import jax
import jax.numpy as jnp
import numpy as np
from jax.experimental import pallas as pl
from jax.experimental.pallas import tpu as pltpu

_C_NUM = 196
_H = 224
_W = 224
_B = 8
_NUM_ITERS = 50
_GRAD_WEIGHT = 10.0
_COLOR_WEIGHT = 10.0
_NEIGH = 10
_SIDE = 2 * _NEIGH
_SLAB = 32
_DIRS = ((-1, 0), (1, 0), (0, -1), (0, 1))


def _centroid_tables():
    num_cols = int(math.sqrt(_C_NUM * _W / _H))
    num_rows = int(math.ceil(_C_NUM / num_cols))
    gy = _H / num_rows
    gx = _W / num_cols
    cents = []
    for i in range(num_rows):
        for j in range(num_cols):
            if len(cents) >= _C_NUM:
                break
            cents.append((int((i + 0.5) * gy), int((j + 0.5) * gx)))
        if len(cents) >= _C_NUM:
            break
    ys = np.array([c[0] for c in cents], np.int32)
    xs = np.array([c[1] for c in cents], np.int32)
    y0 = np.maximum(0, ys - _NEIGH)
    y1 = np.minimum(_H, ys + _NEIGH)
    x0 = np.maximum(0, xs - _NEIGH)
    x1 = np.minimum(_W, xs + _NEIGH)
    sy = np.clip(ys - _NEIGH, 0, _H - _SIDE)
    sx = np.clip(xs - _NEIGH, 0, _W - _SIDE)
    base8 = np.minimum((sy // 8), (_H - _SLAB) // 8)
    return np.stack([ys, xs, y0, y1, x0, x1, sy, sx, base8]).astype(np.int32)


def _roll2(a, dy, dx):
    r = a
    if dy != 0:
        r = jnp.roll(r, dy, axis=1)
    if dx != 0:
        r = jnp.roll(r, dx, axis=2)
    return r


def _shift_zero(a, dy, dx):
    r = _roll2(a, -dy, -dx)
    ri = jax.lax.broadcasted_iota(jnp.int32, a.shape, 1)
    ci = jax.lax.broadcasted_iota(jnp.int32, a.shape, 2)
    ok = (ri + dy >= 0) & (ri + dy < _H) & (ci + dx >= 0) & (ci + dx < _W)
    return jnp.where(ok, r, 0.0)


def _voronoi_body(x_ref, gmap_ref, cent_ref, mask_ref,
                  dist_ref, occ_ref, wg0_ref, cds_ref, m16_ref, flag_ref):
    x = x_ref[...]
    gray = 0.2989 * x[:, 0] + 0.587 * x[:, 1] + 0.114 * x[:, 2]
    gray = gray.astype(jnp.bfloat16).astype(jnp.float32)

    shifts = {}
    for dy in (-1, 0, 1):
        for dx in (-1, 0, 1):
            if dy == 0 and dx == 0:
                continue
            shifts[(dy, dx)] = _shift_zero(gray, dy, dx)
    gx = ((shifts[(-1, 1)] - shifts[(-1, -1)])
          + 2.0 * (shifts[(0, 1)] - shifts[(0, -1)])
          + (shifts[(1, 1)] - shifts[(1, -1)]))
    gy = ((shifts[(1, -1)] - shifts[(-1, -1)])
          + 2.0 * (shifts[(1, 0)] - shifts[(-1, 0)])
          + (shifts[(1, 1)] - shifts[(-1, 1)]))
    g = jnp.sqrt(gx * gx + gy * gy + 1e-08)
    gmap_ref[...] = g[:, None]

    g2 = g * g
    wg0_ref[...] = (g2 * g2) * _GRAD_WEIGHT
    for k in range(4):
        dy, dx = _DIRS[k]
        cds = jnp.zeros_like(gray)
        for c in range(3):
            xc = x[:, c]
            cds = cds + jnp.abs(xc - _roll2(xc, dy, dx))
        cds_ref[k] = cds * _COLOR_WEIGHT

    dist_ref[...] = jnp.full((_B, _H, _W), jnp.inf, jnp.float32)
    m16_ref[...] = jnp.full((_B, _H, _W), -1.0, jnp.bfloat16)
    occ_ref[...] = jnp.zeros((_B, _H, _W), jnp.bfloat16)
    cent_ref[...] = jnp.zeros((_B, _C_NUM, 2), jnp.float32)

    tab = _centroid_tables()
    cio = jax.lax.broadcasted_iota(jnp.int32, (_B, 1, 2), 2)
    for i in range(_C_NUM):
        ys, xs, y0, y1, x0, x1, sy, sx, _ = (int(v) for v in tab[:, i])
        h = y1 - y0
        w = x1 - x0
        win = gmap_ref[:, 0, y0:y1, x0:x1]
        occw = occ_ref[:, y0:y1, x0:x1]
        mv = jnp.min(win, axis=(1, 2))
        cand = (win == mv[:, None, None]) & (occw == jnp.bfloat16(0.0))
        ra = jax.lax.broadcasted_iota(jnp.int32, (_B, h, w), 1) + y0
        ca = jax.lax.broadcasted_iota(jnp.int32, (_B, h, w), 2) + x0
        linv = (ra - sy) * _SIDE + (ca - sx)
        lin = jnp.min(jnp.where(cand, linv, _SIDE * _SIDE), axis=(1, 2))
        found = lin < _SIDE * _SIDE
        li = jnp.where(found, lin, 0)
        py = sy + li // _SIDE
        px = sx + li % _SIDE
        pixm = ((ra == py[:, None, None]) & (ca == px[:, None, None])
                & found[:, None, None])
        occ_ref[:, y0:y1, x0:x1] = jnp.where(pixm, jnp.bfloat16(1.0), occw)
        oy = jnp.where(found, py, ys)
        ox = jnp.where(found, px, xs)
        pix2 = (ra == oy[:, None, None]) & (ca == ox[:, None, None])
        m16_ref[:, y0:y1, x0:x1] = jnp.where(pix2, jnp.bfloat16(i),
                                             m16_ref[:, y0:y1, x0:x1])
        dist_ref[:, y0:y1, x0:x1] = jnp.where(pix2, 0.0,
                                              dist_ref[:, y0:y1, x0:x1])
        cent_ref[:, i:i + 1, :] = jnp.where(
            cio == 0, oy.astype(jnp.float32)[:, None, None],
            ox.astype(jnp.float32)[:, None, None])

    flag_ref[0] = jnp.int32(1)

    def prop_step(it, carry):
        @pl.when(flag_ref[0] != 0)
        def _run():
            wg0 = wg0_ref[...]
            acc = None
            for k in range(4):
                dy, dx = _DIRS[k]
                d = dist_ref[...]
                m = m16_ref[...]
                sd = _roll2(d, dy, dx)
                sm = _roll2(m, dy, dx)
                wd = (sd + wg0) + cds_ref[k]
                up = wd < d
                dist_ref[...] = jnp.where(up, wd, d)
                m16_ref[...] = jnp.where(up, sm, m)
                acc = up if acc is None else (acc | up)
            flag_ref[0] = jnp.any(acc).astype(jnp.int32)
        return carry

    jax.lax.fori_loop(0, _NUM_ITERS, prop_step, 0)
    mask_ref[...] = m16_ref[...].astype(jnp.float32)


def kernel(x, W_edge):
    gmap, cent, mask = pl.pallas_call(
        _voronoi_body,
        out_shape=(
            jax.ShapeDtypeStruct((_B, 1, _H, _W), jnp.float32),
            jax.ShapeDtypeStruct((_B, _C_NUM, 2), jnp.float32),
            jax.ShapeDtypeStruct((_B, _H, _W), jnp.float32),
        ),
        in_specs=[
            pl.BlockSpec(memory_space=pltpu.VMEM),
        ],
        out_specs=(
            pl.BlockSpec(memory_space=pltpu.VMEM),
            pl.BlockSpec(memory_space=pltpu.VMEM),
            pl.BlockSpec(memory_space=pltpu.VMEM),
        ),
        scratch_shapes=[
            pltpu.VMEM((_B, _H, _W), jnp.float32),
            pltpu.VMEM((_B, _H, _W), jnp.bfloat16),
            pltpu.VMEM((_B, _H, _W), jnp.float32),
            pltpu.VMEM((4, _B, _H, _W), jnp.float32),
            pltpu.VMEM((_B, _H, _W), jnp.bfloat16),
            pltpu.SMEM((1,), jnp.int32),
        ],
    )(x)
    return (gmap, cent, mask)


# speedup vs baseline: 1.0261x; 1.0261x over previous
import math

import jax
import jax.numpy as jnp
import numpy as np
from jax.experimental import pallas as pl
from jax.experimental.pallas import tpu as pltpu

_C_NUM = 196
_H = 224
_W = 224
_B = 8
_NUM_ITERS = 50
_GRAD_WEIGHT = 10.0
_COLOR_WEIGHT = 10.0
_NEIGH = 10
_SIDE = 2 * _NEIGH
_SLAB = 32
_DIRS = ((-1, 0), (1, 0), (0, -1), (0, 1))


def _centroid_tables():
    num_cols = int(math.sqrt(_C_NUM * _W / _H))
    num_rows = int(math.ceil(_C_NUM / num_cols))
    gy = _H / num_rows
    gx = _W / num_cols
    cents = []
    for i in range(num_rows):
        for j in range(num_cols):
            if len(cents) >= _C_NUM:
                break
            cents.append((int((i + 0.5) * gy), int((j + 0.5) * gx)))
        if len(cents) >= _C_NUM:
            break
    ys = np.array([c[0] for c in cents], np.int32)
    xs = np.array([c[1] for c in cents], np.int32)
    y0 = np.maximum(0, ys - _NEIGH)
    y1 = np.minimum(_H, ys + _NEIGH)
    x0 = np.maximum(0, xs - _NEIGH)
    x1 = np.minimum(_W, xs + _NEIGH)
    sy = np.clip(ys - _NEIGH, 0, _H - _SIDE)
    sx = np.clip(xs - _NEIGH, 0, _W - _SIDE)
    base8 = np.minimum((sy // 8), (_H - _SLAB) // 8)
    return np.stack([ys, xs, y0, y1, x0, x1, sy, sx, base8]).astype(np.int32)


def _roll2(a, dy, dx):
    r = a
    if dy != 0:
        r = jnp.roll(r, dy, axis=1)
    if dx != 0:
        r = jnp.roll(r, dx, axis=2)
    return r


def _shift_zero(a, dy, dx):
    b = a.shape[0]
    r = a
    if dy == 1:
        r = jnp.concatenate([r[:, 1:, :], jnp.zeros((b, 1, _W), r.dtype)], axis=1)
    elif dy == -1:
        r = jnp.concatenate([jnp.zeros((b, 1, _W), r.dtype), r[:, :-1, :]], axis=1)
    if dx == 1:
        r = jnp.concatenate([r[:, :, 1:], jnp.zeros((b, _H, 1), r.dtype)], axis=2)
    elif dx == -1:
        r = jnp.concatenate([jnp.zeros((b, _H, 1), r.dtype), r[:, :, :-1]], axis=2)
    return r


def _voronoi_body(x_ref, gmap_ref, cent_ref, mask_ref,
                  dist_ref, occ_ref, wg0_ref, cds_ref, m16_ref, flag_ref):
    x = x_ref[...]
    gray = 0.2989 * x[:, 0] + 0.587 * x[:, 1] + 0.114 * x[:, 2]
    gray = gray.astype(jnp.bfloat16).astype(jnp.float32)

    shifts = {}
    for dy in (-1, 0, 1):
        for dx in (-1, 0, 1):
            if dy == 0 and dx == 0:
                continue
            shifts[(dy, dx)] = _shift_zero(gray, dy, dx)
    gx = ((shifts[(-1, 1)] - shifts[(-1, -1)])
          + 2.0 * (shifts[(0, 1)] - shifts[(0, -1)])
          + (shifts[(1, 1)] - shifts[(1, -1)]))
    gy = ((shifts[(1, -1)] - shifts[(-1, -1)])
          + 2.0 * (shifts[(1, 0)] - shifts[(-1, 0)])
          + (shifts[(1, 1)] - shifts[(-1, 1)]))
    g = jnp.sqrt(gx * gx + gy * gy + 1e-08)
    gmap_ref[...] = g[:, None]

    g2 = g * g
    wg0_ref[...] = (g2 * g2) * _GRAD_WEIGHT
    cbase = {}
    for k, axis in ((1, 1), (3, 2)):
        cds = jnp.zeros_like(gray)
        for c in range(3):
            xc = x[:, c]
            cds = cds + jnp.abs(xc - jnp.roll(xc, 1, axis=axis))
        cbase[k] = cds
    cds_ref[1] = cbase[1] * _COLOR_WEIGHT
    cds_ref[0] = jnp.roll(cbase[1], -1, axis=1) * _COLOR_WEIGHT
    cds_ref[3] = cbase[3] * _COLOR_WEIGHT
    cds_ref[2] = jnp.roll(cbase[3], -1, axis=2) * _COLOR_WEIGHT

    dist_ref[...] = jnp.full((_B, _H, _W), jnp.inf, jnp.float32)
    m16_ref[...] = jnp.full((_B, _H, _W), -1.0, jnp.bfloat16)
    occ_ref[...] = jnp.zeros((_B, _H, _W), jnp.float32)
    cent_ref[...] = jnp.zeros((_B, _C_NUM, 2), jnp.float32)

    tab = _centroid_tables()
    cio = jax.lax.broadcasted_iota(jnp.int32, (_B, 1, 2), 2)
    for i in range(_C_NUM):
        ys, xs, y0, y1, x0, x1, sy, sx, _ = (int(v) for v in tab[:, i])
        h = y1 - y0
        w = x1 - x0
        win = gmap_ref[:, 0, y0:y1, x0:x1]
        occw = occ_ref[:, y0:y1, x0:x1]
        mv = jnp.min(win, axis=(1, 2))
        cand = (win == mv[:, None, None]) & (occw == 0.0)
        ra = jax.lax.broadcasted_iota(jnp.int32, (_B, h, w), 1) + y0
        ca = jax.lax.broadcasted_iota(jnp.int32, (_B, h, w), 2) + x0
        linv = (ra - sy) * _SIDE + (ca - sx)
        lin = jnp.min(jnp.where(cand, linv, _SIDE * _SIDE), axis=(1, 2))
        found = lin < _SIDE * _SIDE
        li = jnp.where(found, lin, 0)
        py = sy + li // _SIDE
        px = sx + li % _SIDE
        pixm = ((ra == py[:, None, None]) & (ca == px[:, None, None])
                & found[:, None, None])
        occ_ref[:, y0:y1, x0:x1] = jnp.where(pixm, 1.0, occw)
        oy = jnp.where(found, py, ys)
        ox = jnp.where(found, px, xs)
        pix2 = (ra == oy[:, None, None]) & (ca == ox[:, None, None])
        m16_ref[:, y0:y1, x0:x1] = jnp.where(pix2, jnp.bfloat16(i),
                                             m16_ref[:, y0:y1, x0:x1])
        dist_ref[:, y0:y1, x0:x1] = jnp.where(pix2, 0.0,
                                              dist_ref[:, y0:y1, x0:x1])
        cent_ref[:, i:i + 1, :] = jnp.where(
            cio == 0, oy.astype(jnp.float32)[:, None, None],
            ox.astype(jnp.float32)[:, None, None])

    flag_ref[0] = jnp.int32(1)

    def prop_step(it, carry):
        @pl.when(flag_ref[0] != 0)
        def _run():
            wg0 = wg0_ref[...]
            acc = None
            for k in range(4):
                dy, dx = _DIRS[k]
                d = dist_ref[...]
                m = m16_ref[...]
                sd = _roll2(d, dy, dx)
                sm = _roll2(m, dy, dx)
                wd = (sd + wg0) + cds_ref[k]
                up = wd < d
                dist_ref[...] = jnp.where(up, wd, d)
                m16_ref[...] = jnp.where(up, sm, m)
                acc = up if acc is None else (acc | up)
            flag_ref[0] = jnp.any(acc).astype(jnp.int32)
        return carry

    jax.lax.fori_loop(0, _NUM_ITERS, prop_step, 0)
    mask_ref[...] = m16_ref[...].astype(jnp.float32)


def kernel(x, W_edge):
    gmap, cent, mask = pl.pallas_call(
        _voronoi_body,
        out_shape=(
            jax.ShapeDtypeStruct((_B, 1, _H, _W), jnp.float32),
            jax.ShapeDtypeStruct((_B, _C_NUM, 2), jnp.float32),
            jax.ShapeDtypeStruct((_B, _H, _W), jnp.float32),
        ),
        in_specs=[
            pl.BlockSpec(memory_space=pltpu.VMEM),
        ],
        out_specs=(
            pl.BlockSpec(memory_space=pltpu.VMEM),
            pl.BlockSpec(memory_space=pltpu.VMEM),
            pl.BlockSpec(memory_space=pltpu.VMEM),
        ),
        scratch_shapes=[
            pltpu.VMEM((_B, _H, _W), jnp.float32),
            pltpu.VMEM((_B, _H, _W), jnp.float32),
            pltpu.VMEM((_B, _H, _W), jnp.float32),
            pltpu.VMEM((4, _B, _H, _W), jnp.float32),
            pltpu.VMEM((_B, _H, _W), jnp.bfloat16),
            pltpu.SMEM((1,), jnp.int32),
        ],
    )(x)
    return (gmap, cent, mask)
